# baseline (speedup 1.0000x reference)
import functools

import jax
import jax.numpy as jnp
from jax import lax
from jax.experimental import pallas as pl
from jax.experimental.pallas import tpu as pltpu
from jax.experimental.pallas import tpu_sc as plsc

N = 3200000
NUM_SEGMENTS = 16
NC, NS, L = 2, 16, 16
NW = NC * NS
ROWS_PER_W = N // NW
CH = 10000
NCHUNK = ROWS_PER_W // CH
GROUPS = CH // L
UNROLL = 25
ACC = NUM_SEGMENTS * 3 * L
NF = 5
NFIFTH = N // NF
RSPLIT = 5120
NBLK = NFIFTH // RSPLIT

_mesh = plsc.VectorSubcoreMesh(core_axis_name="c", subcore_axis_name="s")


def _split_body(*refs):
    p_refs, out_refs = refs[:NF], refs[NF:]
    for f in range(NF):
        t = jnp.transpose(p_refs[f][...])
        for c in range(3):
            out_refs[3 * f + c][...] = t[c]


def _split_tc(pos):
    plane = jax.ShapeDtypeStruct((NFIFTH,), jnp.float32)
    return pl.pallas_call(
        _split_body,
        grid=(NBLK,),
        in_specs=[pl.BlockSpec((RSPLIT, 3), lambda i, f=f: (i + f * NBLK, 0))
                  for f in range(NF)],
        out_specs=[pl.BlockSpec((RSPLIT,), lambda i: (i,))] * (3 * NF),
        out_shape=[plane] * (3 * NF),
    )(*([pos] * NF))


@functools.partial(
    pl.kernel,
    mesh=_mesh,
    out_type=jax.ShapeDtypeStruct((NW, ACC), jnp.float32),
    scratch_types=[
        pltpu.VMEM((CH,), jnp.float32),
        pltpu.VMEM((CH,), jnp.float32),
        pltpu.VMEM((CH,), jnp.float32),
        pltpu.VMEM((CH,), jnp.int32),
        pltpu.VMEM((L,), jnp.int32),
        pltpu.VMEM((L,), jnp.int32),
        pltpu.VMEM((ACC,), jnp.float32),
    ],
)
def _seg_max_sc(*refs):
    planes_hbm = refs[:3 * NF]
    ids_hbm = refs[3 * NF]
    out_hbm = refs[3 * NF + 1]
    x_v, y_v, z_v, ids_v, fid_v, lid_v, acc_v = refs[3 * NF + 2:]
    wid = lax.axis_index("s") * NC + lax.axis_index("c")
    neg_inf = jnp.full((L,), -jnp.inf, dtype=jnp.float32)
    planes_v = (x_v, y_v, z_v)

    for i in range(ACC // L):
        acc_v[pl.ds(i * L, L)] = neg_inf

    def chunk_body(t, _):
        r0 = pl.multiple_of(wid * ROWS_PER_W + t * CH, 8)
        pltpu.sync_copy(ids_hbm.at[pl.ds(r0, L)], fid_v)
        pltpu.sync_copy(ids_hbm.at[pl.ds(r0 + CH - L, L)], lid_v)
        f = r0 // NFIFTH
        rloc = pl.multiple_of(r0 - f * NFIFTH, 8)
        for k in range(NF):
            @pl.when(f == k)
            def _copy(k=k):
                for c in range(3):
                    pltpu.sync_copy(planes_hbm[3 * k + c].at[pl.ds(rloc, CH)],
                                    planes_v[c])
        s0 = fid_v[...][0]
        s1 = lid_v[...][L - 1]

        def acc_update(s, rs):
            for c in range(3):
                off = s * (3 * L) + c * L
                acc_v[pl.ds(off, L)] = jnp.maximum(acc_v[pl.ds(off, L)], rs[c])

        @pl.when(s0 == s1)
        def _fast():
            def f_body(it, carry):
                rs = list(carry)
                base = it * (UNROLL * L)
                for u in range(UNROLL):
                    for c in range(3):
                        v = planes_v[c][pl.ds(base + u * L, L)]
                        rs[c] = jnp.maximum(rs[c], v)
                return tuple(rs)

            rs = lax.fori_loop(0, GROUPS // UNROLL, f_body,
                               (neg_inf, neg_inf, neg_inf))
            acc_update(s0, rs)

        @pl.when(s0 != s1)
        def _slow():
            pltpu.sync_copy(ids_hbm.at[pl.ds(r0, CH)], ids_v)

            for s in range(NUM_SEGMENTS):
                @pl.when((s >= s0) & (s <= s1))
                def _sweep(s=s):
                    def g_body(g, carry):
                        id16 = ids_v[pl.ds(g * L, L)]
                        m = id16 == s
                        rs = []
                        for c in range(3):
                            v = planes_v[c][pl.ds(g * L, L)]
                            rs.append(jnp.maximum(carry[c],
                                                  jnp.where(m, v, -jnp.inf)))
                        return tuple(rs)

                    rs = lax.fori_loop(0, GROUPS, g_body,
                                       (neg_inf, neg_inf, neg_inf))
                    acc_update(s, rs)

        return 0

    lax.fori_loop(0, NCHUNK, chunk_body, 0)
    pltpu.sync_copy(acc_v, out_hbm.at[wid])


def _merge_body(parts_ref, out_ref):
    m = jnp.max(parts_ref[...], axis=0, keepdims=True)
    lane = lax.broadcasted_iota(jnp.int32, (NUM_SEGMENTS, ACC), 1)
    srow = lax.broadcasted_iota(jnp.int32, (NUM_SEGMENTS, ACC), 0)
    seg_ok = (lane // (3 * L)) == srow
    comp = (lane % (3 * L)) // L
    mb = jnp.broadcast_to(m, (NUM_SEGMENTS, ACC))
    cols = []
    for c in range(3):
        sel = jnp.where(seg_ok & (comp == c), mb, -jnp.inf)
        cols.append(jnp.max(sel, axis=1, keepdims=True))
    cols.append(jnp.zeros((NUM_SEGMENTS, 128 - 3), jnp.float32))
    out_ref[...] = jnp.concatenate(cols, axis=1)


def kernel(pos, batch, W, b):
    del W, b
    planes = _split_tc(pos)
    ids = batch.astype(jnp.int32)
    parts = _seg_max_sc(*planes, ids)
    padded = pl.pallas_call(
        _merge_body,
        out_shape=jax.ShapeDtypeStruct((NUM_SEGMENTS, 128), jnp.float32),
    )(parts)
    x = padded[:, :3]
    new_pos = jnp.zeros((x.shape[0], 6), dtype=pos.dtype)
    new_batch = jnp.arange(x.shape[0], dtype=jnp.int64)
    return (x, new_pos, new_batch)


# speedup vs baseline: 1.0100x; 1.0100x over previous
import functools

import jax
import jax.numpy as jnp
from jax import lax
from jax.experimental import pallas as pl
from jax.experimental.pallas import tpu as pltpu
from jax.experimental.pallas import tpu_sc as plsc

N = 3200000
NUM_SEGMENTS = 16
NC, NS, L = 2, 16, 16
NW = NC * NS
ROWS_PER_W = N // NW
CH = 10000
NCHUNK = ROWS_PER_W // CH
GROUPS = CH // L
UNROLL = 25
ACC = NUM_SEGMENTS * 3 * L
RSPLIT = 25600

_mesh = plsc.VectorSubcoreMesh(core_axis_name="c", subcore_axis_name="s")


def _split_body(p_ref, x_ref, y_ref, z_ref):
    t = jnp.transpose(p_ref[...])
    x_ref[...] = t[0]
    y_ref[...] = t[1]
    z_ref[...] = t[2]


def _split_tc(pos):
    plane = jax.ShapeDtypeStruct((N,), jnp.float32)
    return pl.pallas_call(
        _split_body,
        grid=(N // RSPLIT,),
        in_specs=[pl.BlockSpec((RSPLIT, 3), lambda i: (i, 0))],
        out_specs=[pl.BlockSpec((RSPLIT,), lambda i: (i,))] * 3,
        out_shape=[plane, plane, plane],
    )(pos)


@functools.partial(
    pl.kernel,
    mesh=_mesh,
    out_type=jax.ShapeDtypeStruct((NW, ACC), jnp.float32),
    scratch_types=[
        pltpu.VMEM((CH,), jnp.float32),
        pltpu.VMEM((CH,), jnp.float32),
        pltpu.VMEM((CH,), jnp.float32),
        pltpu.VMEM((CH,), jnp.int32),
        pltpu.VMEM((L,), jnp.int32),
        pltpu.VMEM((L,), jnp.int32),
        pltpu.VMEM((ACC,), jnp.float32),
    ],
)
def _seg_max_sc(xs_hbm, ys_hbm, zs_hbm, ids_hbm, out_hbm,
                x_v, y_v, z_v, ids_v, fid_v, lid_v, acc_v):
    wid = lax.axis_index("s") * NC + lax.axis_index("c")
    neg_inf = jnp.full((L,), -jnp.inf, dtype=jnp.float32)
    planes = (x_v, y_v, z_v)

    for i in range(ACC // L):
        acc_v[pl.ds(i * L, L)] = neg_inf

    def chunk_body(t, _):
        r0 = pl.multiple_of(wid * ROWS_PER_W + t * CH, 8)
        pltpu.sync_copy(ids_hbm.at[pl.ds(r0, L)], fid_v)
        pltpu.sync_copy(ids_hbm.at[pl.ds(r0 + CH - L, L)], lid_v)
        pltpu.sync_copy(xs_hbm.at[pl.ds(r0, CH)], x_v)
        pltpu.sync_copy(ys_hbm.at[pl.ds(r0, CH)], y_v)
        pltpu.sync_copy(zs_hbm.at[pl.ds(r0, CH)], z_v)
        s0 = fid_v[...][0]
        s1 = lid_v[...][L - 1]

        def acc_update(s, rs):
            for c in range(3):
                off = s * (3 * L) + c * L
                acc_v[pl.ds(off, L)] = jnp.maximum(acc_v[pl.ds(off, L)], rs[c])

        @pl.when(s0 == s1)
        def _fast():
            def f_body(it, carry):
                rs = list(carry)
                base = it * (UNROLL * L)
                for u in range(UNROLL):
                    for c in range(3):
                        v = planes[c][pl.ds(base + u * L, L)]
                        rs[c] = jnp.maximum(rs[c], v)
                return tuple(rs)

            rs = lax.fori_loop(0, GROUPS // UNROLL, f_body,
                               (neg_inf, neg_inf, neg_inf))
            acc_update(s0, rs)

        @pl.when(s0 != s1)
        def _slow():
            pltpu.sync_copy(ids_hbm.at[pl.ds(r0, CH)], ids_v)

            for s in range(NUM_SEGMENTS):
                @pl.when((s >= s0) & (s <= s1))
                def _sweep(s=s):
                    def g_body(g, carry):
                        id16 = ids_v[pl.ds(g * L, L)]
                        m = id16 == s
                        rs = []
                        for c in range(3):
                            v = planes[c][pl.ds(g * L, L)]
                            rs.append(jnp.maximum(carry[c],
                                                  jnp.where(m, v, -jnp.inf)))
                        return tuple(rs)

                    rs = lax.fori_loop(0, GROUPS, g_body,
                                       (neg_inf, neg_inf, neg_inf))
                    acc_update(s, rs)

        return 0

    lax.fori_loop(0, NCHUNK, chunk_body, 0)
    pltpu.sync_copy(acc_v, out_hbm.at[wid])


def _merge_body(parts_ref, out_ref):
    m = jnp.max(parts_ref[...], axis=0, keepdims=True)
    lane = lax.broadcasted_iota(jnp.int32, (NUM_SEGMENTS, ACC), 1)
    srow = lax.broadcasted_iota(jnp.int32, (NUM_SEGMENTS, ACC), 0)
    seg_ok = (lane // (3 * L)) == srow
    comp = (lane % (3 * L)) // L
    mb = jnp.broadcast_to(m, (NUM_SEGMENTS, ACC))
    cols = []
    for c in range(3):
        sel = jnp.where(seg_ok & (comp == c), mb, -jnp.inf)
        cols.append(jnp.max(sel, axis=1, keepdims=True))
    cols.append(jnp.zeros((NUM_SEGMENTS, 128 - 3), jnp.float32))
    out_ref[...] = jnp.concatenate(cols, axis=1)


def kernel(pos, batch, W, b):
    del W, b
    xs, ys, zs = _split_tc(pos)
    ids = batch.astype(jnp.int32)
    parts = _seg_max_sc(xs, ys, zs, ids)
    padded = pl.pallas_call(
        _merge_body,
        out_shape=jax.ShapeDtypeStruct((NUM_SEGMENTS, 128), jnp.float32),
    )(parts)
    x = padded[:, :3]
    new_pos = jnp.zeros((x.shape[0], 6), dtype=pos.dtype)
    new_batch = jnp.arange(x.shape[0], dtype=jnp.int64)
    return (x, new_pos, new_batch)
